# baseline (speedup 1.0000x reference)
import jax
import jax.numpy as jnp
from jax.experimental import pallas as pl
from jax.experimental.pallas import tpu as pltpu

G = 200
Gp = 256
A = 20000
BA = 512
NB = 40
Ap = BA * NB
IOU_THRESHOLD = 0.5
BIG = 2**30


def _stage1_body(anch_ref, boxes_ref, iou_out, idx_out, gbest_out,
                 acc_iou, acc_idx):
    j = pl.program_id(0)
    al = anch_ref[:, 0:1]
    at = anch_ref[:, 1:2]
    ar = anch_ref[:, 2:3]
    ab = anch_ref[:, 3:4]
    bl = boxes_ref[0:1, :]
    bt = boxes_ref[1:2, :]
    br = boxes_ref[2:3, :]
    bb = boxes_ref[3:4, :]

    w = jnp.maximum(jnp.minimum(br, ar) - jnp.maximum(bl, al), 0.0)
    h = jnp.maximum(jnp.minimum(bb, ab) - jnp.maximum(bt, at), 0.0)
    inter = w * h
    a1 = (br - bl) * (bb - bt)
    a2 = (ar - al) * (ab - at)
    iou = inter / (a1 + a2 - inter)

    gi = jax.lax.broadcasted_iota(jnp.int32, (BA, Gp), 1)
    ai = jax.lax.broadcasted_iota(jnp.int32, (BA, Gp), 0) + j * BA

    m = jnp.max(iou, axis=1, keepdims=True)
    amin = jnp.min(jnp.where(iou == m, gi, BIG), axis=1, keepdims=True)
    iou_out[:, :] = m
    idx_out[:, :] = amin

    cmax = jnp.max(iou, axis=0, keepdims=True)
    cidx = jnp.min(jnp.where(iou == cmax, ai, BIG), axis=0, keepdims=True)

    @pl.when(j == 0)
    def _():
        acc_iou[0:1, :] = jnp.full((1, Gp), -1.0, jnp.float32)

    prev_i = acc_iou[0:1, :]
    upd = cmax > prev_i
    acc_iou[0:1, :] = jnp.where(upd, cmax, prev_i)
    @pl.when(j == 0)
    def _():
        acc_idx[0:1, :] = cidx
    @pl.when(j > 0)
    def _():
        acc_idx[0:1, :] = jnp.where(upd, cidx, acc_idx[0:1, :])

    @pl.when(j == NB - 1)
    def _():
        lane = jax.lax.broadcasted_iota(jnp.int32, (1, Gp), 1)
        gbest_out[:, :] = jnp.where(lane < G, acc_idx[0:1, :], -1)


def _stage2_body(iou_ref, idx_ref, gbest_ref, table_ref, anch_ref,
                 bbox_out, lab_out):
    j = pl.program_id(0)
    ai = jax.lax.broadcasted_iota(jnp.int32, (BA, Gp), 0) + j * BA
    gi = jax.lax.broadcasted_iota(jnp.int32, (BA, Gp), 1)

    gb = gbest_ref[0:1, :]
    eqf = gb == ai
    forced_g = jnp.max(jnp.where(eqf, gi, -1), axis=1, keepdims=True)
    forced = forced_g >= 0
    final_g = jnp.where(forced, forced_g, idx_ref[:, :])
    mask = forced | (iou_ref[:, :] > IOU_THRESHOLD)

    onehot = (gi == final_g).astype(jnp.float32)
    gath = jax.lax.dot_general(
        onehot, table_ref[:, :], (((1,), (0,)), ((), ())),
        preferred_element_type=jnp.float32,
        precision=jax.lax.Precision.HIGHEST)

    L = jnp.where(mask, gath[:, 0:1], anch_ref[:, 0:1])
    T = jnp.where(mask, gath[:, 1:2], anch_ref[:, 1:2])
    R = jnp.where(mask, gath[:, 2:3], anch_ref[:, 2:3])
    B = jnp.where(mask, gath[:, 3:4], anch_ref[:, 3:4])
    bbox_out[:, 0:1] = 0.5 * (L + R)
    bbox_out[:, 1:2] = 0.5 * (T + B)
    bbox_out[:, 2:3] = R - L
    bbox_out[:, 3:4] = B - T
    lab = jnp.floor(gath[:, 4:5] + 0.5).astype(jnp.int32)
    lab_out[:, :] = jnp.where(mask, lab, 0)


@jax.jit
def _run(image, boxes, labels, anchors):
    f32 = jnp.float32
    boxes = boxes.astype(f32)
    anchors = anchors.astype(f32)
    anch_p = jnp.zeros((Ap, 4), f32).at[:A].set(anchors)
    boxes_t = jnp.zeros((4, Gp), f32).at[:, :G].set(boxes.T)
    table = (jnp.zeros((Gp, 8), f32)
             .at[:G, 0:4].set(boxes)
             .at[:G, 4].set(labels.astype(f32)))

    iou_b, idx_b, gbest = pl.pallas_call(
        _stage1_body,
        grid=(NB,),
        in_specs=[
            pl.BlockSpec((BA, 4), lambda j: (j, 0)),
            pl.BlockSpec((4, Gp), lambda j: (0, 0)),
        ],
        out_specs=[
            pl.BlockSpec((BA, 1), lambda j: (j, 0)),
            pl.BlockSpec((BA, 1), lambda j: (j, 0)),
            pl.BlockSpec((1, Gp), lambda j: (0, 0)),
        ],
        out_shape=[
            jax.ShapeDtypeStruct((Ap, 1), f32),
            jax.ShapeDtypeStruct((Ap, 1), jnp.int32),
            jax.ShapeDtypeStruct((1, Gp), jnp.int32),
        ],
        scratch_shapes=[
            pltpu.VMEM((8, Gp), f32),
            pltpu.VMEM((8, Gp), jnp.int32),
        ],
    )(anch_p, boxes_t)

    bbox, lab = pl.pallas_call(
        _stage2_body,
        grid=(NB,),
        in_specs=[
            pl.BlockSpec((BA, 1), lambda j: (j, 0)),
            pl.BlockSpec((BA, 1), lambda j: (j, 0)),
            pl.BlockSpec((1, Gp), lambda j: (0, 0)),
            pl.BlockSpec((Gp, 8), lambda j: (0, 0)),
            pl.BlockSpec((BA, 4), lambda j: (j, 0)),
        ],
        out_specs=[
            pl.BlockSpec((BA, 4), lambda j: (j, 0)),
            pl.BlockSpec((BA, 1), lambda j: (j, 0)),
        ],
        out_shape=[
            jax.ShapeDtypeStruct((Ap, 4), f32),
            jax.ShapeDtypeStruct((Ap, 1), jnp.int32),
        ],
    )(iou_b, idx_b, gbest, table, anch_p)

    bboxes_out = bbox[:A]
    labels_out = lab[:A, 0]
    return (image, bboxes_out, labels_out)


def kernel(image, boxes, labels, anchors):
    return _run(image, boxes, labels, anchors)


# speedup vs baseline: 2.0268x; 2.0268x over previous
import jax
import jax.numpy as jnp
from jax.experimental import pallas as pl
from jax.experimental.pallas import tpu as pltpu

G = 200
Gp = 256
A = 20000
BA = 512
NB = 40
Ap = BA * NB
IOU_THRESHOLD = 0.5
BIG = 2**30


def _body(boxes_ref, anch_ref, table_ref, bbox_out, lab_out,
          biou_s, bidx_s, acc_iou, acc_idx):
    p = pl.program_id(0)
    j = pl.program_id(1)

    @pl.when(p == 0)
    def _stage1():
        bl = boxes_ref[:, 0:1]
        bt = boxes_ref[:, 1:2]
        br = boxes_ref[:, 2:3]
        bb = boxes_ref[:, 3:4]
        al = anch_ref[0:1, :]
        at = anch_ref[1:2, :]
        ar = anch_ref[2:3, :]
        ab = anch_ref[3:4, :]

        w = jnp.maximum(jnp.minimum(br, ar) - jnp.maximum(bl, al), 0.0)
        h = jnp.maximum(jnp.minimum(bb, ab) - jnp.maximum(bt, at), 0.0)
        inter = w * h
        a1 = (br - bl) * (bb - bt)
        a2 = (ar - al) * (ab - at)
        iou = inter / (a1 + a2 - inter)

        gi = jax.lax.broadcasted_iota(jnp.int32, (Gp, BA), 0)
        ai = jax.lax.broadcasted_iota(jnp.int32, (Gp, BA), 1) + j * BA

        m = jnp.max(iou, axis=0, keepdims=True)
        amin = jnp.min(jnp.where(iou == m, gi, BIG), axis=0, keepdims=True)
        biou_s[0:1, pl.ds(j * BA, BA)] = m
        bidx_s[0:1, pl.ds(j * BA, BA)] = amin

        rmax = jnp.max(iou, axis=1, keepdims=True)
        ridx = jnp.min(jnp.where(iou == rmax, ai, BIG), axis=1, keepdims=True)

        @pl.when(j == 0)
        def _():
            acc_iou[:, 0:1] = jnp.full((Gp, 1), -1.0, jnp.float32)

        prev_i = acc_iou[:, 0:1]
        upd = rmax > prev_i
        acc_iou[:, 0:1] = jnp.where(upd, rmax, prev_i)

        @pl.when(j == 0)
        def _():
            acc_idx[:, 0:1] = ridx

        @pl.when(j > 0)
        def _():
            acc_idx[:, 0:1] = jnp.where(upd, ridx, acc_idx[:, 0:1])

    @pl.when(p == 1)
    def _stage2():
        ai = jax.lax.broadcasted_iota(jnp.int32, (Gp, BA), 1) + j * BA
        gi = jax.lax.broadcasted_iota(jnp.int32, (Gp, BA), 0)

        row = jax.lax.broadcasted_iota(jnp.int32, (Gp, 1), 0)
        gb = jnp.where(row < G, acc_idx[:, 0:1], -1)
        eqf = gb == ai
        forced_g = jnp.max(jnp.where(eqf, gi, -1), axis=0, keepdims=True)
        forced = forced_g >= 0
        bidx = bidx_s[0:1, pl.ds(j * BA, BA)]
        biou = biou_s[0:1, pl.ds(j * BA, BA)]
        final_g = jnp.where(forced, forced_g, bidx)
        mask = forced | (biou > IOU_THRESHOLD)

        onehot = (gi == final_g).astype(jnp.float32)
        gath = jax.lax.dot_general(
            table_ref[:, :], onehot, (((1,), (0,)), ((), ())),
            preferred_element_type=jnp.float32,
            precision=jax.lax.Precision.HIGHEST)

        al = anch_ref[0:1, :]
        at = anch_ref[1:2, :]
        ar = anch_ref[2:3, :]
        ab = anch_ref[3:4, :]
        L = jnp.where(mask, gath[0:1, :], al)
        T = jnp.where(mask, gath[1:2, :], at)
        R = jnp.where(mask, gath[2:3, :], ar)
        B = jnp.where(mask, gath[3:4, :], ab)
        bbox_out[0:1, :] = 0.5 * (L + R)
        bbox_out[1:2, :] = 0.5 * (T + B)
        bbox_out[2:3, :] = R - L
        bbox_out[3:4, :] = B - T
        lab = jnp.floor(gath[4:5, :] + 0.5).astype(jnp.int32)
        lab_out[0:1, :] = jnp.where(mask, lab, 0)


@jax.jit
def _run(image, boxes, labels, anchors):
    f32 = jnp.float32
    boxes = boxes.astype(f32)
    anchors = anchors.astype(f32)
    boxes_p = jnp.zeros((Gp, 4), f32).at[:G].set(boxes)
    anch_t = jnp.zeros((4, Ap), f32).at[:, :A].set(anchors.T)
    table_t = (jnp.zeros((8, Gp), f32)
               .at[0:4, :G].set(boxes.T)
               .at[4, :G].set(labels.astype(f32)))

    bbox_t, lab = pl.pallas_call(
        _body,
        grid=(2, NB),
        in_specs=[
            pl.BlockSpec((Gp, 4), lambda p, j: (0, 0)),
            pl.BlockSpec((4, BA), lambda p, j: (0, j)),
            pl.BlockSpec((8, Gp), lambda p, j: (0, 0)),
        ],
        out_specs=[
            pl.BlockSpec((4, BA), lambda p, j: (0, j)),
            pl.BlockSpec((1, BA), lambda p, j: (0, j)),
        ],
        out_shape=[
            jax.ShapeDtypeStruct((4, Ap), f32),
            jax.ShapeDtypeStruct((1, Ap), jnp.int32),
        ],
        scratch_shapes=[
            pltpu.VMEM((1, Ap), f32),
            pltpu.VMEM((1, Ap), jnp.int32),
            pltpu.VMEM((Gp, 128), f32),
            pltpu.VMEM((Gp, 128), jnp.int32),
        ],
    )(boxes_p, anch_t, table_t)

    bboxes_out = bbox_t[:, :A].T
    labels_out = lab[0, :A]
    return (image, bboxes_out, labels_out)


def kernel(image, boxes, labels, anchors):
    return _run(image, boxes, labels, anchors)


# speedup vs baseline: 2.3843x; 1.1764x over previous
import jax
import jax.numpy as jnp
from jax.experimental import pallas as pl
from jax.experimental.pallas import tpu as pltpu

G = 200
A = 20000
BA = 512
NB = 40
Ap = BA * NB
IOU_THRESHOLD = 0.5
BIG = 2**30


def _body(boxes_ref, anch_ref, table_ref, bbox_out, lab_out,
          biou_s, bidx_s, acc_iou, acc_idx, gcol_s, gb_s):
    p = pl.program_id(0)
    j = pl.program_id(1)

    @pl.when((p == 0) & (j == 0))
    def _hoist():
        ones = jnp.ones((G, BA), jnp.float32)
        bl = boxes_ref[:, 0:1] * ones
        bt = boxes_ref[:, 1:2] * ones
        br = boxes_ref[:, 2:3] * ones
        bb = boxes_ref[:, 3:4] * ones
        gcol_s[0] = bl
        gcol_s[1] = bt
        gcol_s[2] = br
        gcol_s[3] = bb
        gcol_s[4] = (br - bl) * (bb - bt)

    @pl.when(p == 0)
    def _stage1():
        bl = gcol_s[0]
        bt = gcol_s[1]
        br = gcol_s[2]
        bb = gcol_s[3]
        a1 = gcol_s[4]
        al = anch_ref[0:1, :]
        at = anch_ref[1:2, :]
        ar = anch_ref[2:3, :]
        ab = anch_ref[3:4, :]

        w = jnp.maximum(jnp.minimum(br, ar) - jnp.maximum(bl, al), 0.0)
        h = jnp.maximum(jnp.minimum(bb, ab) - jnp.maximum(bt, at), 0.0)
        inter = w * h
        a2 = (ar - al) * (ab - at)
        iou = inter / (a1 + a2 - inter)

        gi = jax.lax.broadcasted_iota(jnp.int32, (G, BA), 0)
        ai = jax.lax.broadcasted_iota(jnp.int32, (G, BA), 1) + j * BA

        m = jnp.max(iou, axis=0, keepdims=True)
        amin = jnp.min(jnp.where(iou == m, gi, BIG), axis=0, keepdims=True)
        biou_s[0:1, pl.ds(j * BA, BA)] = m
        bidx_s[0:1, pl.ds(j * BA, BA)] = amin

        rmax = jnp.max(iou, axis=1, keepdims=True)
        ridx = jnp.min(jnp.where(iou == rmax, ai, BIG), axis=1, keepdims=True)

        @pl.when(j == 0)
        def _():
            acc_iou[:, 0:1] = jnp.full((G, 1), -1.0, jnp.float32)

        prev_i = acc_iou[:, 0:1]
        upd = rmax > prev_i
        acc_iou[:, 0:1] = jnp.where(upd, rmax, prev_i)

        @pl.when(j == 0)
        def _():
            acc_idx[:, 0:1] = ridx

        @pl.when(j > 0)
        def _():
            acc_idx[:, 0:1] = jnp.where(upd, ridx, acc_idx[:, 0:1])

        @pl.when(j == NB - 1)
        def _():
            gb_s[:, :] = acc_idx[:, 0:1] * jnp.ones((G, BA), jnp.int32)

    @pl.when(p == 1)
    def _stage2():
        ai = jax.lax.broadcasted_iota(jnp.int32, (G, BA), 1) + j * BA
        gi = jax.lax.broadcasted_iota(jnp.int32, (G, BA), 0)

        eqf = gb_s[:, :] == ai
        forced_g = jnp.max(jnp.where(eqf, gi, -1), axis=0, keepdims=True)
        forced = forced_g >= 0
        bidx = bidx_s[0:1, pl.ds(j * BA, BA)]
        biou = biou_s[0:1, pl.ds(j * BA, BA)]
        final_g = jnp.where(forced, forced_g, bidx)
        mask = forced | (biou > IOU_THRESHOLD)

        onehot = (gi == final_g).astype(jnp.float32)
        gath = jax.lax.dot_general(
            table_ref[:, :], onehot, (((1,), (0,)), ((), ())),
            preferred_element_type=jnp.float32,
            precision=jax.lax.Precision.HIGHEST)

        al = anch_ref[0:1, :]
        at = anch_ref[1:2, :]
        ar = anch_ref[2:3, :]
        ab = anch_ref[3:4, :]
        L = jnp.where(mask, gath[0:1, :], al)
        T = jnp.where(mask, gath[1:2, :], at)
        R = jnp.where(mask, gath[2:3, :], ar)
        B = jnp.where(mask, gath[3:4, :], ab)
        bbox_out[0:1, :] = 0.5 * (L + R)
        bbox_out[1:2, :] = 0.5 * (T + B)
        bbox_out[2:3, :] = R - L
        bbox_out[3:4, :] = B - T
        lab = jnp.floor(gath[4:5, :] + 0.5).astype(jnp.int32)
        lab_out[0:1, :] = jnp.where(mask, lab, 0)


@jax.jit
def _run(image, boxes, labels, anchors):
    f32 = jnp.float32
    boxes = boxes.astype(f32)
    anchors = anchors.astype(f32)
    anch_t = jnp.zeros((4, Ap), f32).at[:, :A].set(anchors.T)
    table_t = (jnp.zeros((8, G), f32)
               .at[0:4, :].set(boxes.T)
               .at[4, :].set(labels.astype(f32)))

    bbox_t, lab = pl.pallas_call(
        _body,
        grid=(2, NB),
        in_specs=[
            pl.BlockSpec((G, 4), lambda p, j: (0, 0)),
            pl.BlockSpec((4, BA), lambda p, j: (0, j)),
            pl.BlockSpec((8, G), lambda p, j: (0, 0)),
        ],
        out_specs=[
            pl.BlockSpec((4, BA), lambda p, j: (0, j)),
            pl.BlockSpec((1, BA), lambda p, j: (0, j)),
        ],
        out_shape=[
            jax.ShapeDtypeStruct((4, Ap), f32),
            jax.ShapeDtypeStruct((1, Ap), jnp.int32),
        ],
        scratch_shapes=[
            pltpu.VMEM((1, Ap), f32),
            pltpu.VMEM((1, Ap), jnp.int32),
            pltpu.VMEM((G, 128), f32),
            pltpu.VMEM((G, 128), jnp.int32),
            pltpu.VMEM((5, G, BA), f32),
            pltpu.VMEM((G, BA), jnp.int32),
        ],
    )(boxes, anch_t, table_t)

    bboxes_out = bbox_t[:, :A].T
    labels_out = lab[0, :A]
    return (image, bboxes_out, labels_out)


def kernel(image, boxes, labels, anchors):
    return _run(image, boxes, labels, anchors)


# speedup vs baseline: 3.2073x; 1.3451x over previous
import jax
import jax.numpy as jnp
from jax.experimental import pallas as pl
from jax.experimental.pallas import tpu as pltpu

G = 200
A = 20000
BA = 1024
NB = 20
Ap = BA * NB
IOU_THRESHOLD = 0.5
BIG = 2**30


def _body(boxes_ref, anch_ref, table_ref, bbox_out, lab_out,
          biou_s, bidx_s, acc_iou, acc_idx, gcol_s, gb_s):
    p = pl.program_id(0)
    j = pl.program_id(1)

    @pl.when((p == 0) & (j == 0))
    def _hoist():
        ones = jnp.ones((G, BA), jnp.float32)
        bl = boxes_ref[:, 0:1] * ones
        bt = boxes_ref[:, 1:2] * ones
        br = boxes_ref[:, 2:3] * ones
        bb = boxes_ref[:, 3:4] * ones
        gcol_s[0] = bl
        gcol_s[1] = bt
        gcol_s[2] = br
        gcol_s[3] = bb
        gcol_s[4] = (br - bl) * (bb - bt)

    @pl.when(p == 0)
    def _stage1():
        bl = gcol_s[0]
        bt = gcol_s[1]
        br = gcol_s[2]
        bb = gcol_s[3]
        a1 = gcol_s[4]
        al = anch_ref[0:1, :]
        at = anch_ref[1:2, :]
        ar = anch_ref[2:3, :]
        ab = anch_ref[3:4, :]

        w = jnp.maximum(jnp.minimum(br, ar) - jnp.maximum(bl, al), 0.0)
        h = jnp.maximum(jnp.minimum(bb, ab) - jnp.maximum(bt, at), 0.0)
        inter = w * h
        a2 = (ar - al) * (ab - at)
        iou = inter / (a1 + a2 - inter)

        gi = jax.lax.broadcasted_iota(jnp.int32, (G, BA), 0)
        ai = jax.lax.broadcasted_iota(jnp.int32, (G, BA), 1) + j * BA

        m = jnp.max(iou, axis=0, keepdims=True)
        amin = jnp.min(jnp.where(iou == m, gi, BIG), axis=0, keepdims=True)
        biou_s[0:1, pl.ds(j * BA, BA)] = m
        bidx_s[0:1, pl.ds(j * BA, BA)] = amin

        rmax = jnp.max(iou, axis=1, keepdims=True)
        ridx = jnp.min(jnp.where(iou == rmax, ai, BIG), axis=1, keepdims=True)

        @pl.when(j == 0)
        def _():
            acc_iou[:, 0:1] = jnp.full((G, 1), -1.0, jnp.float32)

        prev_i = acc_iou[:, 0:1]
        upd = rmax > prev_i
        acc_iou[:, 0:1] = jnp.where(upd, rmax, prev_i)

        @pl.when(j == 0)
        def _():
            acc_idx[:, 0:1] = ridx

        @pl.when(j > 0)
        def _():
            acc_idx[:, 0:1] = jnp.where(upd, ridx, acc_idx[:, 0:1])

        @pl.when(j == NB - 1)
        def _():
            gb_s[:, :] = acc_idx[:, 0:1] * jnp.ones((G, BA), jnp.int32)

    @pl.when(p == 1)
    def _stage2():
        ai = jax.lax.broadcasted_iota(jnp.int32, (G, BA), 1) + j * BA
        gi = jax.lax.broadcasted_iota(jnp.int32, (G, BA), 0)

        eqf = gb_s[:, :] == ai
        forced_g = jnp.max(jnp.where(eqf, gi, -1), axis=0, keepdims=True)
        forced = forced_g >= 0
        bidx = bidx_s[0:1, pl.ds(j * BA, BA)]
        biou = biou_s[0:1, pl.ds(j * BA, BA)]
        final_g = jnp.where(forced, forced_g, bidx)
        mask = forced | (biou > IOU_THRESHOLD)

        onehot = (gi == final_g).astype(jnp.float32)
        gath = jax.lax.dot_general(
            table_ref[:, :], onehot, (((1,), (0,)), ((), ())),
            preferred_element_type=jnp.float32,
            precision=jax.lax.Precision.HIGHEST)

        al = anch_ref[0:1, :]
        at = anch_ref[1:2, :]
        ar = anch_ref[2:3, :]
        ab = anch_ref[3:4, :]
        L = jnp.where(mask, gath[0:1, :], al)
        T = jnp.where(mask, gath[1:2, :], at)
        R = jnp.where(mask, gath[2:3, :], ar)
        B = jnp.where(mask, gath[3:4, :], ab)
        bbox_out[0:1, :] = 0.5 * (L + R)
        bbox_out[1:2, :] = 0.5 * (T + B)
        bbox_out[2:3, :] = R - L
        bbox_out[3:4, :] = B - T
        lab = jnp.floor(gath[4:5, :] + 0.5).astype(jnp.int32)
        lab_out[0:1, :] = jnp.where(mask, lab, 0)


@jax.jit
def _run(image, boxes, labels, anchors):
    f32 = jnp.float32
    boxes = boxes.astype(f32)
    anchors = anchors.astype(f32)
    anch_t = jnp.zeros((4, Ap), f32).at[:, :A].set(anchors.T)
    table_t = (jnp.zeros((8, G), f32)
               .at[0:4, :].set(boxes.T)
               .at[4, :].set(labels.astype(f32)))

    bbox_t, lab = pl.pallas_call(
        _body,
        grid=(2, NB),
        in_specs=[
            pl.BlockSpec((G, 4), lambda p, j: (0, 0)),
            pl.BlockSpec((4, BA), lambda p, j: (0, j)),
            pl.BlockSpec((8, G), lambda p, j: (0, 0)),
        ],
        out_specs=[
            pl.BlockSpec((4, BA), lambda p, j: (0, j)),
            pl.BlockSpec((1, BA), lambda p, j: (0, j)),
        ],
        out_shape=[
            jax.ShapeDtypeStruct((4, Ap), f32),
            jax.ShapeDtypeStruct((1, Ap), jnp.int32),
        ],
        scratch_shapes=[
            pltpu.VMEM((1, Ap), f32),
            pltpu.VMEM((1, Ap), jnp.int32),
            pltpu.VMEM((G, 128), f32),
            pltpu.VMEM((G, 128), jnp.int32),
            pltpu.VMEM((5, G, BA), f32),
            pltpu.VMEM((G, BA), jnp.int32),
        ],
    )(boxes, anch_t, table_t)

    bboxes_out = bbox_t[:, :A].T
    labels_out = lab[0, :A]
    return (image, bboxes_out, labels_out)


def kernel(image, boxes, labels, anchors):
    return _run(image, boxes, labels, anchors)


# speedup vs baseline: 3.5907x; 1.1196x over previous
import jax
import jax.numpy as jnp
from jax.experimental import pallas as pl
from jax.experimental.pallas import tpu as pltpu

G = 200
A = 20000
BA = 2048
NB = 10
Ap = BA * NB
IOU_THRESHOLD = 0.5
BIG = 2**30


def _body(boxes_ref, anch_ref, table_ref, bbox_out, lab_out,
          biou_s, bidx_s, acc_iou, acc_idx, gcol_s, gb_s):
    p = pl.program_id(0)
    j = pl.program_id(1)

    @pl.when((p == 0) & (j == 0))
    def _hoist():
        ones = jnp.ones((G, BA), jnp.float32)
        bl = boxes_ref[:, 0:1] * ones
        bt = boxes_ref[:, 1:2] * ones
        br = boxes_ref[:, 2:3] * ones
        bb = boxes_ref[:, 3:4] * ones
        gcol_s[0] = bl
        gcol_s[1] = bt
        gcol_s[2] = br
        gcol_s[3] = bb
        gcol_s[4] = (br - bl) * (bb - bt)

    @pl.when(p == 0)
    def _stage1():
        bl = gcol_s[0]
        bt = gcol_s[1]
        br = gcol_s[2]
        bb = gcol_s[3]
        a1 = gcol_s[4]
        al = anch_ref[0:1, :]
        at = anch_ref[1:2, :]
        ar = anch_ref[2:3, :]
        ab = anch_ref[3:4, :]

        w = jnp.maximum(jnp.minimum(br, ar) - jnp.maximum(bl, al), 0.0)
        h = jnp.maximum(jnp.minimum(bb, ab) - jnp.maximum(bt, at), 0.0)
        inter = w * h
        a2 = (ar - al) * (ab - at)
        iou = inter / (a1 + a2 - inter)

        gi = jax.lax.broadcasted_iota(jnp.int32, (G, BA), 0)
        ai = jax.lax.broadcasted_iota(jnp.int32, (G, BA), 1) + j * BA

        m = jnp.max(iou, axis=0, keepdims=True)
        amin = jnp.min(jnp.where(iou == m, gi, BIG), axis=0, keepdims=True)
        biou_s[0:1, pl.ds(j * BA, BA)] = m
        bidx_s[0:1, pl.ds(j * BA, BA)] = amin

        rmax = jnp.max(iou, axis=1, keepdims=True)
        ridx = jnp.min(jnp.where(iou == rmax, ai, BIG), axis=1, keepdims=True)

        @pl.when(j == 0)
        def _():
            acc_iou[:, 0:1] = jnp.full((G, 1), -1.0, jnp.float32)

        prev_i = acc_iou[:, 0:1]
        upd = rmax > prev_i
        acc_iou[:, 0:1] = jnp.where(upd, rmax, prev_i)

        @pl.when(j == 0)
        def _():
            acc_idx[:, 0:1] = ridx

        @pl.when(j > 0)
        def _():
            acc_idx[:, 0:1] = jnp.where(upd, ridx, acc_idx[:, 0:1])

        @pl.when(j == NB - 1)
        def _():
            gb_s[:, :] = acc_idx[:, 0:1] * jnp.ones((G, BA), jnp.int32)

    @pl.when(p == 1)
    def _stage2():
        ai = jax.lax.broadcasted_iota(jnp.int32, (G, BA), 1) + j * BA
        gi = jax.lax.broadcasted_iota(jnp.int32, (G, BA), 0)

        eqf = gb_s[:, :] == ai
        forced_g = jnp.max(jnp.where(eqf, gi, -1), axis=0, keepdims=True)
        forced = forced_g >= 0
        bidx = bidx_s[0:1, pl.ds(j * BA, BA)]
        biou = biou_s[0:1, pl.ds(j * BA, BA)]
        final_g = jnp.where(forced, forced_g, bidx)
        mask = forced | (biou > IOU_THRESHOLD)

        onehot = (gi == final_g).astype(jnp.float32)
        gath = jax.lax.dot_general(
            table_ref[:, :], onehot, (((1,), (0,)), ((), ())),
            preferred_element_type=jnp.float32,
            precision=jax.lax.Precision.HIGHEST)

        al = anch_ref[0:1, :]
        at = anch_ref[1:2, :]
        ar = anch_ref[2:3, :]
        ab = anch_ref[3:4, :]
        L = jnp.where(mask, gath[0:1, :], al)
        T = jnp.where(mask, gath[1:2, :], at)
        R = jnp.where(mask, gath[2:3, :], ar)
        B = jnp.where(mask, gath[3:4, :], ab)
        bbox_out[0:1, :] = 0.5 * (L + R)
        bbox_out[1:2, :] = 0.5 * (T + B)
        bbox_out[2:3, :] = R - L
        bbox_out[3:4, :] = B - T
        lab = jnp.floor(gath[4:5, :] + 0.5).astype(jnp.int32)
        lab_out[0:1, :] = jnp.where(mask, lab, 0)


@jax.jit
def _run(image, boxes, labels, anchors):
    f32 = jnp.float32
    boxes = boxes.astype(f32)
    anchors = anchors.astype(f32)
    anch_t = jnp.zeros((4, Ap), f32).at[:, :A].set(anchors.T)
    table_t = (jnp.zeros((8, G), f32)
               .at[0:4, :].set(boxes.T)
               .at[4, :].set(labels.astype(f32)))

    bbox_t, lab = pl.pallas_call(
        _body,
        grid=(2, NB),
        in_specs=[
            pl.BlockSpec((G, 4), lambda p, j: (0, 0)),
            pl.BlockSpec((4, BA), lambda p, j: (0, j)),
            pl.BlockSpec((8, G), lambda p, j: (0, 0)),
        ],
        out_specs=[
            pl.BlockSpec((4, BA), lambda p, j: (0, j)),
            pl.BlockSpec((1, BA), lambda p, j: (0, j)),
        ],
        out_shape=[
            jax.ShapeDtypeStruct((4, Ap), f32),
            jax.ShapeDtypeStruct((1, Ap), jnp.int32),
        ],
        scratch_shapes=[
            pltpu.VMEM((1, Ap), f32),
            pltpu.VMEM((1, Ap), jnp.int32),
            pltpu.VMEM((G, 128), f32),
            pltpu.VMEM((G, 128), jnp.int32),
            pltpu.VMEM((5, G, BA), f32),
            pltpu.VMEM((G, BA), jnp.int32),
        ],
    )(boxes, anch_t, table_t)

    bboxes_out = bbox_t[:, :A].T
    labels_out = lab[0, :A]
    return (image, bboxes_out, labels_out)


def kernel(image, boxes, labels, anchors):
    return _run(image, boxes, labels, anchors)
